# Initial kernel scaffold; baseline (speedup 1.0000x reference)
import jax
import jax.numpy as jnp
from jax.experimental import pallas as pl


def kernel(x, W1, b1, Wm, bm, Wout, bout, alpha, edge_attr, edge_index):
    raise NotImplementedError("write your pallas kernel here")


# speedup vs baseline: 3.3789x; 3.3789x over previous
import functools

import jax
import jax.numpy as jnp
from jax import lax
from jax.experimental import pallas as pl
from jax.experimental.pallas import tpu as pltpu
from jax.experimental.pallas import tpu_sc as plsc

N = 10000
E = 320000
F = 128
H = 128
C = 32
L = 3

NW = 32
CH = 80
SPC = 25
EW = E // NW
NCH = E // CH
CPW = EW // CH
NST = CPW // SPC
NSUB = 16
RPS = N // NSUB
ZR = 125

_mesh = plsc.VectorSubcoreMesh(core_axis_name="c", subcore_axis_name="s")


def _make_prop(nlv, accumulate):
    nout = 1 if accumulate else nlv

    @functools.partial(
        pl.kernel,
        out_type=jax.ShapeDtypeStruct((nout, 2, N, H), jnp.float32),
        mesh=_mesh,
        compiler_params=pltpu.CompilerParams(
            use_tc_tiling_on_sc=False, needs_layout_passes=False),
        scratch_types=[
            pltpu.VMEM((SPC, CH), jnp.int32),
            pltpu.VMEM((SPC, CH), jnp.int32),
            pltpu.VMEM((SPC, CH), jnp.float32),
            pltpu.VMEM((CH, H), jnp.float32),
            pltpu.VMEM((ZR, H), jnp.float32),
            pltpu.VMEM_SHARED((N, H), jnp.float32),
            pltpu.SemaphoreType.DMA,
        ],
    )
    def prop(tab_hbm, src_hbm, dst_hbm, ea_hbm, out_hbm,
             src_v, dst_v, ea_v, rows_v, zbuf, h_sh, sem):
        cid = lax.axis_index("c")
        sid = lax.axis_index("s")
        wid = sid * 2 + cid

        def zrow(r, carry):
            for v in range(H // 16):
                zbuf[r, pl.ds(v * 16, 16)] = jnp.zeros((16,), jnp.float32)
            return carry
        lax.fori_loop(0, ZR, zrow, 0)

        for i in range(nlv):
            if (i == 0) or (not accumulate):
                for zc in range(RPS // ZR):
                    pltpu.sync_copy(
                        zbuf, h_sh.at[pl.ds(sid * RPS + zc * ZR, ZR), :])
                plsc.subcore_barrier()

            for st in range(NST):
                row0 = wid * CPW + st * SPC
                pltpu.sync_copy(src_hbm.at[i, pl.ds(row0, SPC), :], src_v)
                pltpu.sync_copy(dst_hbm.at[i, pl.ds(row0, SPC), :], dst_v)
                pltpu.sync_copy(ea_hbm.at[i, pl.ds(row0, SPC), :], ea_v)
                def chunk(j, carry):
                    pltpu.async_copy(
                        tab_hbm.at[src_v.at[j]], rows_v, sem).wait()
                    j16 = jnp.broadcast_to(j, (16,)).astype(jnp.int32)

                    def ebody(e, c2):
                        e16 = jnp.broadcast_to(e, (16,)).astype(jnp.int32)
                        eab = plsc.load_gather(ea_v, [j16, e16])
                        for v in range(H // 16):
                            rows_v[e, pl.ds(v * 16, 16)] = (
                                rows_v[e, pl.ds(v * 16, 16)] * eab)
                        return c2
                    lax.fori_loop(0, CH, ebody, 0)
                    pltpu.sync_copy(rows_v, h_sh.at[dst_v.at[j]], add=True)
                    return carry
                lax.fori_loop(0, SPC, chunk, 0)

            if (i == nlv - 1) or (not accumulate):
                plsc.subcore_barrier()
                oi = 0 if accumulate else i
                pltpu.sync_copy(
                    h_sh.at[pl.ds(sid * RPS, RPS), :],
                    out_hbm.at[oi, cid, pl.ds(sid * RPS, RPS), :])
                if (not accumulate) and (i < nlv - 1):
                    plsc.subcore_barrier()

    return prop


_prop_multi = _make_prop(L, accumulate=False)
_prop_acc = _make_prop(L, accumulate=True)


_RB = 2000


def _lin1(x, W1, b1):
    def body(x_ref, w_ref, b_ref, o_ref):
        o_ref[...] = lax.dot_general(
            x_ref[...], w_ref[...], (((1,), (1,)), ((), ())),
            preferred_element_type=jnp.float32) + b_ref[...]

    return pl.pallas_call(
        body,
        grid=(N // _RB,),
        in_specs=[
            pl.BlockSpec((_RB, F), lambda i: (i, 0)),
            pl.BlockSpec((H, F), lambda i: (0, 0)),
            pl.BlockSpec((1, H), lambda i: (0, 0)),
        ],
        out_specs=pl.BlockSpec((_RB, H), lambda i: (i, 0)),
        out_shape=jax.ShapeDtypeStruct((N, H), jnp.float32),
    )(x, W1, b1)


def _mix(hp, Wl, bl):
    def body(h_ref, w_ref, b_ref, o_ref):
        h = h_ref[0, 0] + h_ref[0, 1]
        h = jnp.where(h > 0, h, jnp.exp(h) - 1.0)
        o_ref[...] = (lax.dot_general(
            h, w_ref[0], (((1,), (1,)), ((), ())),
            preferred_element_type=jnp.float32) + b_ref[0])[None]

    return pl.pallas_call(
        body,
        grid=(L, N // _RB),
        in_specs=[
            pl.BlockSpec((1, 2, _RB, H), lambda i, r: (i, 0, r, 0)),
            pl.BlockSpec((1, H, H), lambda i, r: (i, 0, 0)),
            pl.BlockSpec((1, 1, H), lambda i, r: (i, 0, 0)),
        ],
        out_specs=pl.BlockSpec((1, _RB, H), lambda i, r: (i, r, 0)),
        out_shape=jax.ShapeDtypeStruct((L, N, H), jnp.float32),
    )(hp, Wl, bl)


def _upd(x, sp, a2):
    def body(a_ref, x_ref, s_ref, o_ref):
        sa = 1.0 / (1.0 + jnp.exp(-a_ref[0, 0]))
        o_ref[...] = x_ref[...] + sa * (s_ref[0] + s_ref[1])

    return pl.pallas_call(
        body,
        grid=(N // _RB,),
        in_specs=[
            pl.BlockSpec(memory_space=pltpu.SMEM),
            pl.BlockSpec((_RB, H), lambda i: (i, 0)),
            pl.BlockSpec((2, _RB, H), lambda i: (0, i, 0)),
        ],
        out_specs=pl.BlockSpec((_RB, H), lambda i: (i, 0)),
        out_shape=jax.ShapeDtypeStruct((N, H), jnp.float32),
    )(a2, x, sp)


def _head(x, Wout, bout):
    def body(x_ref, w_ref, b_ref, o_ref):
        y = lax.dot_general(
            jnp.maximum(x_ref[...], 0.0), w_ref[...], (((1,), (1,)), ((), ())),
            preferred_element_type=jnp.float32) + b_ref[...]
        m = jnp.max(y, axis=1, keepdims=True)
        o_ref[...] = (y - m) - jnp.log(
            jnp.sum(jnp.exp(y - m), axis=1, keepdims=True))

    return pl.pallas_call(
        body,
        grid=(N // _RB,),
        in_specs=[
            pl.BlockSpec((_RB, H), lambda i: (i, 0)),
            pl.BlockSpec((C, H), lambda i: (0, 0)),
            pl.BlockSpec((1, C), lambda i: (0, 0)),
        ],
        out_specs=pl.BlockSpec((_RB, C), lambda i: (i, 0)),
        out_shape=jax.ShapeDtypeStruct((N, C), jnp.float32),
    )(x, Wout, bout)


def kernel(x, W1, b1, Wm, bm, Wout, bout, alpha, edge_attr, edge_index):
    src3 = edge_index[:, 0, :].reshape(L, NCH, CH)
    dst3 = edge_index[:, 1, :].reshape(L, NCH, CH)
    ea3 = edge_attr.reshape(L, NCH, CH)
    off = (jnp.arange(L, dtype=jnp.int32) * N)[:, None, None]
    srco3 = src3 + off
    a2 = alpha.reshape(1, 1)

    h = _lin1(x, W1, b1.reshape(1, H))
    for l in range(3):
        hp = _prop_multi(h, src3, dst3, ea3)
        m = _mix(hp, Wm[l], bm[l].reshape(L, 1, H))
        sp = _prop_acc(m.reshape(L * N, H), srco3, dst3, ea3)
        h = _upd(h, sp[0], a2)
    return _head(h, Wout, bout.reshape(1, C))
